# baseline (speedup 1.0000x reference)
import functools

import jax
import jax.numpy as jnp
from jax.experimental import pallas as pl
from jax.experimental.pallas import tpu as pltpu
from jax.experimental.pallas import tpu_sc as plsc

_HIDDEN = 64
_PAD = 128
_CHUNK = 256
_MM_BLOCK = 8192


def _sc_gather(table128, idx_flat):
    n = idx_flat.shape[0]
    d = table128.shape[1]
    mesh = plsc.VectorSubcoreMesh(core_axis_name="core", subcore_axis_name="subcore")
    nw = 32
    b_per_w = n // nw
    n_chunks = b_per_w // _CHUNK

    @functools.partial(
        pl.kernel,
        out_type=jax.ShapeDtypeStruct((n, d), table128.dtype),
        mesh=mesh,
        scratch_types=[
            pltpu.VMEM((_CHUNK,), jnp.int32),
            pltpu.VMEM((_CHUNK, d), jnp.float32),
            pltpu.SemaphoreType.DMA,
        ],
    )
    def gather_kernel(x_hbm, i_hbm, o_hbm, idx_v, rows_v, sem):
        wid = jax.lax.axis_index("subcore") * 2 + jax.lax.axis_index("core")
        base = wid * b_per_w

        @pl.loop(0, n_chunks)
        def _(c):
            off = base + c * _CHUNK
            pltpu.sync_copy(i_hbm.at[pl.ds(off, _CHUNK)], idx_v)
            pltpu.async_copy(x_hbm.at[idx_v], rows_v, sem).wait()
            pltpu.sync_copy(rows_v, o_hbm.at[pl.ds(off, _CHUNK)])

    return gather_kernel(table128, idx_flat)


def _tc_project(x, W2, b):
    n, d_in = x.shape
    d_out = W2.shape[1]

    def mm_kernel(x_ref, w_ref, b_ref, o_ref):
        o_ref[...] = (
            jnp.dot(x_ref[...], w_ref[...], preferred_element_type=jnp.float32)
            + b_ref[...]
        )

    return pl.pallas_call(
        mm_kernel,
        grid=(n // _MM_BLOCK,),
        in_specs=[
            pl.BlockSpec((_MM_BLOCK, d_in), lambda i: (i, 0)),
            pl.BlockSpec((d_in, d_out), lambda i: (0, 0)),
            pl.BlockSpec((1, d_out), lambda i: (0, 0)),
        ],
        out_specs=pl.BlockSpec((_MM_BLOCK, d_out), lambda i: (i, 0)),
        out_shape=jax.ShapeDtypeStruct((n, d_out), jnp.float32),
    )(x, W2, b.reshape(1, d_out))


def kernel(input_ids, attention_mask, embedding, W, b):
    del attention_mask
    bsz, seqlen = input_ids.shape
    n = bsz * seqlen
    idx_flat = input_ids.reshape(n)
    vocab, hidden = embedding.shape
    table128 = jnp.pad(embedding, ((0, 0), (0, _PAD - hidden)))
    W2 = jnp.pad(W, ((0, _PAD - hidden), (0, 0)))
    gathered = _sc_gather(table128, idx_flat)
    out = _tc_project(gathered, W2, b)
    return out.reshape(bsz, seqlen, _HIDDEN)
